# Initial kernel scaffold; baseline (speedup 1.0000x reference)
import jax
import jax.numpy as jnp
from jax.experimental import pallas as pl


def kernel(x, perm_indices):
    raise NotImplementedError("write your pallas kernel here")


# speedup vs baseline: 2.5198x; 2.5198x over previous
import functools

import jax
import jax.numpy as jnp
from jax import lax
from jax.experimental import pallas as pl
from jax.experimental.pallas import tpu as pltpu
from jax.experimental.pallas import tpu_sc as plsc

_B, _S, _D = 4, 4096, 2048
_R = _B * _S
_NW = 32
_ROWS_PER_W = _R // _NW
_CHUNK = 32
_NCHUNK = _ROWS_PER_W // _CHUNK


def _permute_rows(x2d, perm32):
    mesh = plsc.VectorSubcoreMesh(core_axis_name="c", subcore_axis_name="s")

    @functools.partial(
        pl.kernel,
        mesh=mesh,
        out_type=jax.ShapeDtypeStruct((_R, _D), jnp.float32),
        scratch_types=[
            pltpu.VMEM((_CHUNK,), jnp.int32),
            pltpu.VMEM((_CHUNK, _D), jnp.float32),
            pltpu.SemaphoreType.DMA,
        ],
    )
    def k(x_hbm, perm_hbm, out_hbm, idx_v, rows_v, sem):
        wid = lax.axis_index("s") * 2 + lax.axis_index("c")
        base = wid * _ROWS_PER_W
        row_off = (base // _S) * _S
        pbase = base - row_off
        for c in range(_NCHUNK):
            pltpu.sync_copy(perm_hbm.at[pl.ds(pbase + c * _CHUNK, _CHUNK)],
                            idx_v)
            for i in range(_CHUNK // 16):
                idx_v[pl.ds(i * 16, 16)] = idx_v[pl.ds(i * 16, 16)] + row_off
            pltpu.async_copy(x_hbm.at[idx_v], rows_v, sem).wait()
            pltpu.sync_copy(rows_v, out_hbm.at[pl.ds(base + c * _CHUNK,
                                                     _CHUNK)])

    return k(x2d, perm32)


def kernel(x, perm_indices):
    x2d = x.reshape(_R, _D)
    perm32 = perm_indices.astype(jnp.int32)
    out = _permute_rows(x2d, perm32)
    return out.reshape(_B, _S, _D)


# speedup vs baseline: 2.7929x; 1.1084x over previous
import functools

import jax
import jax.numpy as jnp
from jax import lax
from jax.experimental import pallas as pl
from jax.experimental.pallas import tpu as pltpu
from jax.experimental.pallas import tpu_sc as plsc

_B, _S, _D = 4, 4096, 2048
_R = _B * _S
_NW = 32
_ROWS_PER_W = _R // _NW
_CHUNK = 16
_NCHUNK = _ROWS_PER_W // _CHUNK
_NBUF = 2


def _permute_rows(x2d, perm32):
    mesh = plsc.VectorSubcoreMesh(core_axis_name="c", subcore_axis_name="s")

    @functools.partial(
        pl.kernel,
        mesh=mesh,
        out_type=jax.ShapeDtypeStruct((_R, _D), jnp.float32),
        scratch_types=(
            [pltpu.VMEM((_ROWS_PER_W,), jnp.int32)]
            + [pltpu.VMEM((_CHUNK, _D), jnp.float32)] * _NBUF
            + [pltpu.SemaphoreType.DMA] * (2 * _NBUF)
        ),
    )
    def k(x_hbm, perm_hbm, out_hbm, idx_all, *bufs_and_sems):
        rows = bufs_and_sems[:_NBUF]
        gsem = bufs_and_sems[_NBUF:2 * _NBUF]
        ssem = bufs_and_sems[2 * _NBUF:]
        wid = lax.axis_index("s") * 2 + lax.axis_index("c")
        base = wid * _ROWS_PER_W
        row_off = (base // _S) * _S
        pbase = base - row_off
        pltpu.sync_copy(perm_hbm.at[pl.ds(pbase, _ROWS_PER_W)], idx_all)

        def gather(c):
            iv = idx_all[pl.ds(c * _CHUNK, _CHUNK)] + row_off
            return pltpu.async_copy(x_hbm.at[iv], rows[c % _NBUF],
                                    gsem[c % _NBUF])

        gh = [None] * _NCHUNK
        sh = [None] * _NCHUNK
        gh[0] = gather(0)
        for c in range(_NCHUNK):
            cur = c % _NBUF
            if c + 1 < _NCHUNK:
                if c + 1 >= _NBUF:
                    sh[c + 1 - _NBUF].wait()
                gh[c + 1] = gather(c + 1)
            gh[c].wait()
            sh[c] = pltpu.async_copy(
                rows[cur], out_hbm.at[pl.ds(base + c * _CHUNK, _CHUNK)],
                ssem[cur])
        for c in range(_NCHUNK - _NBUF, _NCHUNK):
            sh[c].wait()

    return k(x2d, perm32)


def kernel(x, perm_indices):
    x2d = x.reshape(_R, _D)
    perm32 = perm_indices.astype(jnp.int32)
    out = _permute_rows(x2d, perm32)
    return out.reshape(_B, _S, _D)


# speedup vs baseline: 2.8119x; 1.0068x over previous
import functools

import jax
import jax.numpy as jnp
from jax import lax
from jax.experimental import pallas as pl
from jax.experimental.pallas import tpu as pltpu
from jax.experimental.pallas import tpu_sc as plsc

_B, _S, _D = 4, 4096, 2048
_R = _B * _S
_NW = 32
_ROWS_PER_W = _R // _NW
_CHUNK = 16
_NCHUNK = _ROWS_PER_W // _CHUNK
_NBUF = 3


def _permute_rows(x2d, perm32):
    mesh = plsc.VectorSubcoreMesh(core_axis_name="c", subcore_axis_name="s")

    @functools.partial(
        pl.kernel,
        mesh=mesh,
        out_type=jax.ShapeDtypeStruct((_R, _D), jnp.float32),
        scratch_types=(
            [pltpu.VMEM((_ROWS_PER_W,), jnp.int32)]
            + [pltpu.VMEM((_CHUNK, _D), jnp.float32)] * _NBUF
            + [pltpu.SemaphoreType.DMA] * (2 * _NBUF)
        ),
    )
    def k(x_hbm, perm_hbm, out_hbm, idx_all, *bufs_and_sems):
        rows = bufs_and_sems[:_NBUF]
        gsem = bufs_and_sems[_NBUF:2 * _NBUF]
        ssem = bufs_and_sems[2 * _NBUF:]
        wid = lax.axis_index("s") * 2 + lax.axis_index("c")
        base = wid * _ROWS_PER_W
        row_off = (base // _S) * _S
        pbase = base - row_off
        pltpu.sync_copy(perm_hbm.at[pl.ds(pbase, _ROWS_PER_W)], idx_all)

        def gather(c):
            iv = idx_all[pl.ds(c * _CHUNK, _CHUNK)] + row_off
            return pltpu.async_copy(x_hbm.at[iv], rows[c % _NBUF],
                                    gsem[c % _NBUF])

        gh = [None] * _NCHUNK
        sh = [None] * _NCHUNK
        for g in range(min(_NBUF - 1, _NCHUNK)):
            gh[g] = gather(g)
        for c in range(_NCHUNK):
            g = c + _NBUF - 1
            if g < _NCHUNK:
                if g >= _NBUF:
                    sh[g - _NBUF].wait()
                gh[g] = gather(g)
            gh[c].wait()
            sh[c] = pltpu.async_copy(
                rows[c % _NBUF], out_hbm.at[pl.ds(base + c * _CHUNK, _CHUNK)],
                ssem[c % _NBUF])
        for c in range(max(0, _NCHUNK - _NBUF), _NCHUNK):
            sh[c].wait()

    return k(x2d, perm32)


def kernel(x, perm_indices):
    x2d = x.reshape(_R, _D)
    perm32 = perm_indices.astype(jnp.int32)
    out = _permute_rows(x2d, perm32)
    return out.reshape(_B, _S, _D)
